# baseline (speedup 1.0000x reference)
import functools

import jax
import jax.numpy as jnp
from jax import lax
from jax.experimental import pallas as pl
from jax.experimental.pallas import tpu as pltpu
from jax.experimental.pallas import tpu_sc as plsc

B, S, T, M = 32, 512, 2048, 80
NB_SC = 4
B_TC = B - NB_SC
BB = 4
NW = 32
UNITS = NB_SC * 10


def _tc_body(mt_ref, mp_ref, pmp_ref, mw_ref,
             pt_ref, pp_ref, et_ref, ep_ref,
             ldp_ref, dt_ref, pst_ref, psp_ref, sw_ref,
             out_ref, acc1_ref, acc2_ref, sacc_ref):
    i = pl.program_id(0)

    @pl.when(i == 0)
    def _small():
        sw = sw_ref[...]
        n_src = jnp.sum(sw)
        s_pitch = jnp.sum((pp_ref[...] - pt_ref[...]) ** 2 * sw)
        s_energy = jnp.sum((ep_ref[...] - et_ref[...]) ** 2 * sw)
        ldt = jnp.log(dt_ref[...].astype(jnp.float32) + 1.0)
        s_dur = jnp.sum((ldp_ref[...] - ldt) ** 2 * sw)
        psp = psp_ref[...]
        pst = pst_ref[...]
        d = psp - pst
        s_mid = jnp.sum(d * d)
        cond = jnp.logical_and((0.0 * psp) > (psp - 0.5), pst != 0.0)
        s_pen = jnp.sum(cond.astype(jnp.float32))
        sacc_ref[0] = s_pitch
        sacc_ref[1] = s_energy
        sacc_ref[2] = s_dur
        sacc_ref[3] = n_src
        sacc_ref[4] = s_mid
        sacc_ref[5] = s_pen
        sacc_ref[6] = jnp.sum(mw_ref[...].astype(jnp.float32))
        acc1_ref[...] = jnp.zeros_like(acc1_ref)
        acc2_ref[...] = jnp.zeros_like(acc2_ref)

    for j in range(BB):
        onehot = (jax.lax.broadcasted_iota(jnp.int32, (1, B), 1) == i * BB + j
                  ).astype(jnp.bfloat16)
        w2 = jax.lax.dot_general(
            onehot, mw_ref[...], (((1,), (0,)), ((), ())),
            preferred_element_type=jnp.float32)
        mt = mt_ref[j]
        cs1 = jnp.sum(jnp.abs(mp_ref[j] - mt), axis=0, keepdims=True)
        cs2 = jnp.sum(jnp.abs(pmp_ref[j] - mt), axis=0, keepdims=True)
        acc1_ref[...] += cs1 * w2
        acc2_ref[...] += cs2 * w2

    @pl.when(i == B_TC // BB - 1)
    def _final():
        n_src = sacc_ref[3]
        pause_loss = (sacc_ref[4] / (B * S) + 100.0 * 0.5 * sacc_ref[5] / B) / S
        out_ref[0] = jnp.sum(acc1_ref[...])
        out_ref[1] = jnp.sum(acc2_ref[...])
        out_ref[2] = sacc_ref[6] * M
        out_ref[3] = sacc_ref[0] / n_src
        out_ref[4] = sacc_ref[1] / n_src
        out_ref[5] = sacc_ref[2] / n_src
        out_ref[6] = pause_loss * 0.7


def _sc_body(mt_hbm, mp_hbm, pmp_hbm, wx_hbm, out_hbm,
             t_v, p_v, q_v, w_v, o_v, sem):
    wid = lax.axis_index("s") * 2 + lax.axis_index("c")
    acc1 = jnp.zeros((16,), jnp.float32)
    acc2 = jnp.zeros((16,), jnp.float32)
    for rep in range(2):
        u = wid + rep * NW

        @pl.when(u < UNITS)
        def _unit():
            b = B_TC + u // 10
            mi = (u % 10) * 8
            cps = [pltpu.async_copy(mt_hbm.at[b, pl.ds(mi, 8), :], t_v, sem),
                   pltpu.async_copy(mp_hbm.at[b, pl.ds(mi, 8), :], p_v, sem),
                   pltpu.async_copy(pmp_hbm.at[b, pl.ds(mi, 8), :], q_v, sem),
                   pltpu.async_copy(wx_hbm.at[b - B_TC, pl.ds(mi, 8), :], w_v, sem)]
            for cp in cps:
                cp.wait()

        for r in range(8):
            def _col(c, accs):
                a1, a2 = accs
                sl = pl.ds(c * 16, 16)
                mt = t_v[r, sl]
                w = w_v[r, sl]
                a1 = a1 + w * jnp.abs(p_v[r, sl] - mt)
                a2 = a2 + w * jnp.abs(q_v[r, sl] - mt)
                return (a1, a2)
            zero16 = jnp.zeros((16,), jnp.float32)
            p1, p2 = lax.fori_loop(0, 128, _col, (zero16, zero16))
            valid = (u < UNITS).astype(jnp.float32)
            acc1 = acc1 + p1 * valid
            acc2 = acc2 + p2 * valid

    o_v[...] = acc1
    pltpu.sync_copy(o_v, out_hbm.at[0, wid])
    o_v[...] = acc2
    pltpu.sync_copy(o_v, out_hbm.at[1, wid])


def _sc_partial(mt_sc, mp_sc, pmp_sc, wexp):
    mesh = plsc.VectorSubcoreMesh(core_axis_name="c", subcore_axis_name="s")
    f = functools.partial(
        pl.kernel, mesh=mesh,
        out_type=jax.ShapeDtypeStruct((2, NW, 16), jnp.float32),
        scratch_types=[pltpu.VMEM((8, T), jnp.float32),
                       pltpu.VMEM((8, T), jnp.float32),
                       pltpu.VMEM((8, T), jnp.float32),
                       pltpu.VMEM((8, T), jnp.float32),
                       pltpu.VMEM((16,), jnp.float32),
                       pltpu.SemaphoreType.DMA],
    )(_sc_body)
    return f(mt_sc, mp_sc, pmp_sc, wexp)


def kernel(mel_targets, pitch_targets, energy_targets, pause_targets,
           mel_predictions, postnet_mel_predictions, pitch_predictions,
           energy_predictions, log_duration_predictions, pause_predictions,
           duration_targets, src_masks, mel_masks):
    mt3 = jnp.transpose(mel_targets, (0, 2, 1))
    mp3 = jnp.transpose(mel_predictions, (0, 2, 1))
    pmp3 = jnp.transpose(postnet_mel_predictions, (0, 2, 1))
    mw = jnp.logical_not(mel_masks).astype(jnp.bfloat16)
    sw = jnp.logical_not(src_masks).astype(jnp.float32)
    wexp = jnp.broadcast_to(
        jnp.logical_not(mel_masks[B_TC:, None, :]).astype(jnp.float32),
        (NB_SC, M, T))

    sc_out = _sc_partial(mt3, mp3, pmp3, wexp)

    mel_spec = pl.BlockSpec((BB, M, T), lambda i: (i, 0, 0))
    full2d = pl.BlockSpec((B, T), lambda i: (0, 0))
    small_spec = pl.BlockSpec((B, S), lambda i: (0, 0))

    out = pl.pallas_call(
        _tc_body,
        grid=(B_TC // BB,),
        in_specs=[mel_spec, mel_spec, mel_spec, full2d] + [small_spec] * 9,
        out_specs=pl.BlockSpec(memory_space=pltpu.SMEM),
        out_shape=jax.ShapeDtypeStruct((8,), jnp.float32),
        scratch_shapes=[pltpu.VMEM((1, T), jnp.float32),
                        pltpu.VMEM((1, T), jnp.float32),
                        pltpu.SMEM((8,), jnp.float32)],
    )(mt3, mp3, pmp3, mw,
      pitch_targets, pitch_predictions,
      energy_targets, energy_predictions,
      log_duration_predictions, duration_targets,
      pause_targets, pause_predictions, sw)

    sc_sums = jnp.sum(sc_out, axis=(1, 2))
    mel_loss = (out[0] + sc_sums[0]) / out[2]
    postnet_loss = (out[1] + sc_sums[1]) / out[2]
    pitch_loss = out[3]
    energy_loss = out[4]
    dur_loss = out[5]
    pause_w = out[6]
    total = (mel_loss + postnet_loss + dur_loss + pitch_loss +
             energy_loss + pause_w)
    return (total, mel_loss, postnet_loss, pitch_loss, energy_loss,
            dur_loss, pause_w)


# speedup vs baseline: 2.0463x; 2.0463x over previous
import jax
import jax.numpy as jnp
from jax.experimental import pallas as pl
from jax.experimental.pallas import tpu as pltpu

B, S, T, M = 32, 512, 2048, 80
BB = 8


def _body(mt_ref, mp_ref, pmp_ref, mw_ref,
          pt_ref, pp_ref, et_ref, ep_ref,
          ldp_ref, dt_ref, pst_ref, psp_ref, sw_ref,
          out_ref, acc1_ref, acc2_ref, sacc_ref):
    i = pl.program_id(0)

    @pl.when(i == 0)
    def _small():
        sw = sw_ref[...]
        n_src = jnp.sum(sw)
        s_pitch = jnp.sum((pp_ref[...] - pt_ref[...]) ** 2 * sw)
        s_energy = jnp.sum((ep_ref[...] - et_ref[...]) ** 2 * sw)
        ldt = jnp.log(dt_ref[...].astype(jnp.float32) + 1.0)
        s_dur = jnp.sum((ldp_ref[...] - ldt) ** 2 * sw)
        psp = psp_ref[...]
        pst = pst_ref[...]
        d = psp - pst
        s_mid = jnp.sum(d * d)
        cond = jnp.logical_and((0.0 * psp) > (psp - 0.5), pst != 0.0)
        s_pen = jnp.sum(cond.astype(jnp.float32))
        sacc_ref[0] = s_pitch
        sacc_ref[1] = s_energy
        sacc_ref[2] = s_dur
        sacc_ref[3] = n_src
        sacc_ref[4] = s_mid
        sacc_ref[5] = s_pen
        sacc_ref[6] = jnp.sum(mw_ref[...].astype(jnp.float32))
        acc1_ref[...] = jnp.zeros_like(acc1_ref)
        acc2_ref[...] = jnp.zeros_like(acc2_ref)

    for j in range(BB):
        onehot = (jax.lax.broadcasted_iota(jnp.int32, (1, B), 1) == i * BB + j
                  ).astype(jnp.bfloat16)
        w2 = jax.lax.dot_general(
            onehot, mw_ref[...], (((1,), (0,)), ((), ())),
            preferred_element_type=jnp.float32)
        mt = mt_ref[j]
        cs1 = jnp.sum(jnp.abs(mp_ref[j] - mt), axis=0, keepdims=True)
        cs2 = jnp.sum(jnp.abs(pmp_ref[j] - mt), axis=0, keepdims=True)
        acc1_ref[...] += cs1 * w2
        acc2_ref[...] += cs2 * w2

    @pl.when(i == B // BB - 1)
    def _final():
        n_mel = sacc_ref[6] * M
        mel_loss = jnp.sum(acc1_ref[...]) / n_mel
        postnet_loss = jnp.sum(acc2_ref[...]) / n_mel
        n_src = sacc_ref[3]
        pitch_loss = sacc_ref[0] / n_src
        energy_loss = sacc_ref[1] / n_src
        dur_loss = sacc_ref[2] / n_src
        pause_loss = (sacc_ref[4] / (B * S) + 100.0 * 0.5 * sacc_ref[5] / B) / S
        pause_w = pause_loss * 0.7
        out_ref[1] = mel_loss
        out_ref[2] = postnet_loss
        out_ref[3] = pitch_loss
        out_ref[4] = energy_loss
        out_ref[5] = dur_loss
        out_ref[6] = pause_w
        out_ref[0] = (mel_loss + postnet_loss + dur_loss + pitch_loss +
                      energy_loss + pause_w)


def kernel(mel_targets, pitch_targets, energy_targets, pause_targets,
           mel_predictions, postnet_mel_predictions, pitch_predictions,
           energy_predictions, log_duration_predictions, pause_predictions,
           duration_targets, src_masks, mel_masks):
    mt3 = jnp.transpose(mel_targets, (0, 2, 1))
    mp3 = jnp.transpose(mel_predictions, (0, 2, 1))
    pmp3 = jnp.transpose(postnet_mel_predictions, (0, 2, 1))
    mw = jnp.logical_not(mel_masks).astype(jnp.bfloat16)
    sw = jnp.logical_not(src_masks).astype(jnp.float32)

    mel_spec = pl.BlockSpec((BB, M, T), lambda i: (i, 0, 0))
    full2d = pl.BlockSpec((B, T), lambda i: (0, 0))
    small_spec = pl.BlockSpec((B, S), lambda i: (0, 0))

    out = pl.pallas_call(
        _body,
        grid=(B // BB,),
        in_specs=[mel_spec, mel_spec, mel_spec, full2d] + [small_spec] * 9,
        out_specs=pl.BlockSpec(memory_space=pltpu.SMEM),
        out_shape=jax.ShapeDtypeStruct((8,), jnp.float32),
        scratch_shapes=[pltpu.VMEM((1, T), jnp.float32),
                        pltpu.VMEM((1, T), jnp.float32),
                        pltpu.SMEM((8,), jnp.float32)],
    )(mt3, mp3, pmp3, mw,
      pitch_targets, pitch_predictions,
      energy_targets, energy_predictions,
      log_duration_predictions, duration_targets,
      pause_targets, pause_predictions, sw)

    return (out[0], out[1], out[2], out[3], out[4], out[5], out[6])
